# baseline (speedup 1.0000x reference)
import functools

import jax
import jax.numpy as jnp
from jax import lax
from jax.experimental import pallas as pl
from jax.experimental.pallas import tpu as pltpu
from jax.experimental.pallas import tpu_sc as plsc

NC = 2
NS = 16
NW = NC * NS

V = 1000000
D = 64
B = 16384
H = 50

NBLK = (V + 127) // 128
NPAIR = 64 * NBLK
NUNITS = H * (B // 128)
UNITS_PER_W = NUNITS // NW

_mesh = lambda: plsc.VectorSubcoreMesh(core_axis_name="c", subcore_axis_name="s")


def _t1_pairs(tT):
  def body(t_ref, o_ref):
    bt = t_ref[...].T
    o_ref[...] = jnp.concatenate([bt[0:64, :], bt[64:128, :]], axis=1)

  return pl.pallas_call(
      body,
      grid=(NBLK,),
      in_specs=[pl.BlockSpec((64, 128), lambda j: (0, j))],
      out_specs=pl.BlockSpec((64, 128), lambda j: (j, 0)),
      out_shape=jax.ShapeDtypeStruct((NPAIR, 128), jnp.float32),
  )(tT)


def _t2_select(og, xT):
  def body(x_ref, og_ref, o_ref):
    for hl in range(8):
      blk = og_ref[hl, 0]
      par = (x_ref[pl.ds(hl, 1), :] >> 6) & 1
      bt = blk.T
      o_ref[hl] = jnp.where(par == 1, bt[64:128, :], bt[0:64, :])

  return pl.pallas_call(
      body,
      grid=((H + 7) // 8, B // 128),
      in_specs=[
          pl.BlockSpec((8, 128), lambda hb, j: (hb, j)),
          pl.BlockSpec((8, 1, 128, 128), lambda hb, j: (hb, j, 0, 0)),
      ],
      out_specs=pl.BlockSpec((8, D, 128), lambda hb, j: (hb, 0, j)),
      out_shape=jax.ShapeDtypeStruct((H, D, B), jnp.float32),
  )(xT, og)


def _build_gather():
  @functools.partial(
      pl.kernel,
      out_type=jax.ShapeDtypeStruct((H, B // 128, 128, 128), jnp.float32),
      mesh=_mesh(),
      compiler_params=pltpu.CompilerParams(
          use_tc_tiling_on_sc=True, needs_layout_passes=False),
      scratch_types=[
          pltpu.VMEM((2, 8, 128), jnp.int32),
          pltpu.VMEM((2, 128), jnp.int32),
          pltpu.VMEM((2, 128, 128), jnp.float32),
          [pltpu.SemaphoreType.DMA] * 2,
          [pltpu.SemaphoreType.DMA] * 2,
          [pltpu.SemaphoreType.DMA] * 2,
      ],
  )
  def gath_k(xT_hbm, t2_hbm, og_hbm, xt_v, pidx_v, buf_v, xsems, gsems,
             ssems):
    wid = lax.axis_index("s") * NC + lax.axis_index("c")

    def unit_of(s):
      u = wid + NW * s
      return u // 128, u % 128

    def issue_xload(s, b):
      h, j = unit_of(s)
      pltpu.async_copy(
          xT_hbm.at[pl.ds(8 * (h // 8), 8), pl.ds(128 * j, 128)],
          xt_v.at[b], xsems[b])

    def wait_xload(s, b):
      h, j = unit_of(s)
      pltpu.make_async_copy(
          xT_hbm.at[pl.ds(8 * (h // 8), 8), pl.ds(128 * j, 128)],
          xt_v.at[b], xsems[b]).wait()

    def issue_gather(s, b):
      h, _ = unit_of(s)
      hl = h % 8
      for q in range(8):
        idx = xt_v[b, hl, pl.ds(16 * q, 16)]
        pidx_v[b, pl.ds(16 * q, 16)] = ((idx >> 7) << 6) | (
            idx & jnp.full((16,), 63, jnp.int32))
      pltpu.async_copy(t2_hbm.at[pidx_v.at[b]], buf_v.at[b], gsems[b])

    def wait_gather(b):
      pltpu.make_async_copy(
          t2_hbm.at[pidx_v.at[b]], buf_v.at[b], gsems[b]).wait()

    def store(s, b):
      h, j = unit_of(s)
      return pltpu.make_async_copy(buf_v.at[b], og_hbm.at[h, j], ssems[b])

    issue_xload(0, 0)
    wait_xload(0, 0)
    issue_gather(0, 0)
    issue_xload(1, 1)

    def group(g, carry):
      for b in range(2):
        s = 2 * g + b

        @pl.when((s >= 1) & (s + 1 < UNITS_PER_W))
        def _():
          store(s - 1, 1 - b).wait()

        @pl.when(s + 1 < UNITS_PER_W)
        def _():
          wait_xload(s + 1, 1 - b)
          issue_gather(s + 1, 1 - b)

        wait_gather(b)
        store(s, b).start()

        @pl.when(s + 2 < UNITS_PER_W)
        def _():
          issue_xload(s + 2, b)
      return carry

    lax.fori_loop(0, UNITS_PER_W // 2, group, 0)
    store(UNITS_PER_W - 2, 0).wait()
    store(UNITS_PER_W - 1, 1).wait()

  return gath_k


def kernel(x, table):
  tT = table.T
  xT = x.astype(jnp.int32).T
  t2 = _t1_pairs(tT)
  og = _build_gather()(xT, t2)
  o3 = _t2_select(og, xT)
  return o3.transpose(2, 0, 1)


# speedup vs baseline: 2.2520x; 2.2520x over previous
import functools

import jax
import jax.numpy as jnp
from jax import lax
from jax.experimental import pallas as pl
from jax.experimental.pallas import tpu as pltpu
from jax.experimental.pallas import tpu_sc as plsc

NC = 2
NS = 16
NW = NC * NS

V = 1000000
D = 64
B = 16384
H = 50

NBLK = (V + 127) // 128
NPAIR = 64 * NBLK
NUNITS = H * (B // 128)
UNITS_PER_W = NUNITS // NW

_mesh = lambda: plsc.VectorSubcoreMesh(core_axis_name="c", subcore_axis_name="s")


def _eye128():
  r = lax.broadcasted_iota(jnp.int32, (128, 128), 0)
  c = lax.broadcasted_iota(jnp.int32, (128, 128), 1)
  return (r == c).astype(jnp.float32)


def _t1_pairs(tT):
  def body(t_ref, o_ref):
    eye = _eye128()
    for t in range(4):
      blk = t_ref[:, pl.ds(128 * t, 128)]
      bt = lax.dot_general(
          eye, blk, (((0,), (1,)), ((), ())),
          preferred_element_type=jnp.float32)
      o_ref[pl.ds(64 * t, 64), :] = jnp.concatenate(
          [bt[0:64, :], bt[64:128, :]], axis=1)

  return pl.pallas_call(
      body,
      grid=((NBLK + 3) // 4,),
      in_specs=[pl.BlockSpec((64, 512), lambda j: (0, j))],
      out_specs=pl.BlockSpec((256, 128), lambda j: (j, 0)),
      out_shape=jax.ShapeDtypeStruct((NPAIR, 128), jnp.float32),
  )(tT)


def _t2_select(og, xT):
  def body(x_ref, og_ref, o_ref):
    eye = _eye128()
    for hl in range(8):
      blk = og_ref[hl, 0]
      par = (x_ref[pl.ds(hl, 1), :] >> 6) & 1
      bt = lax.dot_general(
          eye, blk, (((0,), (1,)), ((), ())),
          preferred_element_type=jnp.float32)
      o_ref[hl] = jnp.where(par == 1, bt[64:128, :], bt[0:64, :])

  return pl.pallas_call(
      body,
      grid=((H + 7) // 8, B // 128),
      in_specs=[
          pl.BlockSpec((8, 128), lambda hb, j: (hb, j)),
          pl.BlockSpec((8, 1, 128, 128), lambda hb, j: (hb, j, 0, 0)),
      ],
      out_specs=pl.BlockSpec((8, D, 128), lambda hb, j: (hb, 0, j)),
      out_shape=jax.ShapeDtypeStruct((H, D, B), jnp.float32),
  )(xT, og)


def _build_gather():
  @functools.partial(
      pl.kernel,
      out_type=jax.ShapeDtypeStruct((H, B // 128, 128, 128), jnp.float32),
      mesh=_mesh(),
      compiler_params=pltpu.CompilerParams(
          use_tc_tiling_on_sc=True, needs_layout_passes=False),
      scratch_types=[
          pltpu.VMEM((2, 8, 128), jnp.int32),
          pltpu.VMEM((2, 128), jnp.int32),
          pltpu.VMEM((2, 128, 128), jnp.float32),
          [pltpu.SemaphoreType.DMA] * 2,
          [pltpu.SemaphoreType.DMA] * 2,
          [pltpu.SemaphoreType.DMA] * 2,
      ],
  )
  def gath_k(xT_hbm, t2_hbm, og_hbm, xt_v, pidx_v, buf_v, xsems, gsems,
             ssems):
    wid = lax.axis_index("s") * NC + lax.axis_index("c")

    def unit_of(s):
      u = wid + NW * s
      return u // 128, u % 128

    def issue_xload(s, b):
      h, j = unit_of(s)
      pltpu.async_copy(
          xT_hbm.at[pl.ds(8 * (h // 8), 8), pl.ds(128 * j, 128)],
          xt_v.at[b], xsems[b])

    def wait_xload(s, b):
      h, j = unit_of(s)
      pltpu.make_async_copy(
          xT_hbm.at[pl.ds(8 * (h // 8), 8), pl.ds(128 * j, 128)],
          xt_v.at[b], xsems[b]).wait()

    def issue_gather(s, b):
      h, _ = unit_of(s)
      hl = h % 8
      for q in range(8):
        idx = xt_v[b, hl, pl.ds(16 * q, 16)]
        pidx_v[b, pl.ds(16 * q, 16)] = ((idx >> 7) << 6) | (
            idx & jnp.full((16,), 63, jnp.int32))
      pltpu.async_copy(t2_hbm.at[pidx_v.at[b]], buf_v.at[b], gsems[b])

    def wait_gather(b):
      pltpu.make_async_copy(
          t2_hbm.at[pidx_v.at[b]], buf_v.at[b], gsems[b]).wait()

    def store(s, b):
      h, j = unit_of(s)
      return pltpu.make_async_copy(buf_v.at[b], og_hbm.at[h, j], ssems[b])

    issue_xload(0, 0)
    wait_xload(0, 0)
    issue_gather(0, 0)
    issue_xload(1, 1)

    def group(g, carry):
      for b in range(2):
        s = 2 * g + b

        @pl.when((s >= 1) & (s + 1 < UNITS_PER_W))
        def _():
          store(s - 1, 1 - b).wait()

        @pl.when(s + 1 < UNITS_PER_W)
        def _():
          wait_xload(s + 1, 1 - b)
          issue_gather(s + 1, 1 - b)

        wait_gather(b)
        store(s, b).start()

        @pl.when(s + 2 < UNITS_PER_W)
        def _():
          issue_xload(s + 2, b)
      return carry

    lax.fori_loop(0, UNITS_PER_W // 2, group, 0)
    store(UNITS_PER_W - 2, 0).wait()
    store(UNITS_PER_W - 1, 1).wait()

  return gath_k


def kernel(x, table):
  tT = table.T
  xT = x.astype(jnp.int32).T
  t2 = _t1_pairs(tT)
  og = _build_gather()(xT, t2)
  o3 = _t2_select(og, xT)
  return o3.transpose(2, 0, 1)


# speedup vs baseline: 4.0961x; 1.8189x over previous
import functools

import jax
import jax.numpy as jnp
from jax import lax
from jax.experimental import pallas as pl
from jax.experimental.pallas import tpu as pltpu
from jax.experimental.pallas import tpu_sc as plsc

NC = 2
NS = 16
NW = NC * NS
IW = 128
NCH = 2
NBUF = 5


def _build(V, D, R):
  rows_per_w = R // NW
  n_steps = rows_per_w // NCH
  assert n_steps % NBUF == 0 and n_steps >= 2 * NBUF
  mesh = plsc.VectorSubcoreMesh(core_axis_name="c", subcore_axis_name="s")

  @functools.partial(
      pl.kernel,
      out_type=jax.ShapeDtypeStruct((R, IW, D), jnp.float32),
      mesh=mesh,
      compiler_params=pltpu.CompilerParams(use_tc_tiling_on_sc=False),
      scratch_types=[
          pltpu.VMEM((rows_per_w, IW), jnp.int32),
          pltpu.VMEM((NBUF, NCH, IW, D), jnp.float32),
          [pltpu.SemaphoreType.DMA] * NBUF,
          [pltpu.SemaphoreType.DMA] * NBUF,
      ],
  )
  def gather_kernel(x_hbm, tab_hbm, out_hbm, idx_v, rows_v, gsems, ssems):
    wid = lax.axis_index("s") * NC + lax.axis_index("c")
    base = wid * rows_per_w
    pltpu.sync_copy(x_hbm.at[pl.ds(base, rows_per_w)], idx_v)

    def issue_gathers(i, b):
      for j in range(NCH):
        pltpu.async_copy(
            tab_hbm.at[idx_v.at[i * NCH + j]], rows_v.at[b, j], gsems[b])

    def wait_gathers(i, b):
      for j in range(NCH):
        pltpu.make_async_copy(
            tab_hbm.at[idx_v.at[i * NCH + j]], rows_v.at[b, j],
            gsems[b]).wait()

    def store(i, b):
      return pltpu.make_async_copy(
          rows_v.at[b], out_hbm.at[pl.ds(base + i * NCH, NCH)], ssems[b])

    for k in range(NBUF - 1):
      issue_gathers(k, k)

    def group(g, carry):
      for k in range(NBUF):
        i = g * NBUF + k
        bb = (k - 1) % NBUF

        @pl.when(i > 0)
        def _():
          store(i - 1, bb).wait()

        @pl.when(i + NBUF - 1 < n_steps)
        def _():
          issue_gathers(i + NBUF - 1, bb)

        wait_gathers(i, k)
        store(i, k).start()
      return carry

    lax.fori_loop(0, n_steps // NBUF, group, 0)
    store(n_steps - 1, NBUF - 1).wait()

  return gather_kernel


def kernel(x, table):
  B, H = x.shape
  V, D = table.shape
  n = B * H
  assert n % (NW * NCH * IW) == 0
  R = n // IW
  xf = x.reshape(R, IW).astype(jnp.int32)
  out = _build(V, D, R)(xf, table)
  return out.reshape(B, H, D)
